# baseline (speedup 1.0000x reference)
import functools

import jax
import jax.numpy as jnp
from jax import lax
from jax.experimental import pallas as pl
from jax.experimental.pallas import tpu as pltpu
from jax.experimental.pallas import tpu_sc as plsc

N = 10000
D = 128
H = 2
E = 160000
RN = 1000
R = 3
NLAYERS = 2

LANES = 16
NTILES = 32
ROWS = 10112
NPB = ROWS // NTILES
PROC = 80
CAPP = 5120
EPPT = 4992
DIVM = 26547
DIVS = 23


def _hal_body(x_ref, w_ref, asrc_ref, adst_ref, h_ref, al_ref):
    x = x_ref[...]
    for h in range(H):
        hh = jnp.dot(x, w_ref[h], preferred_element_type=jnp.float32)
        h_ref[:, h * D:(h + 1) * D] = hh
        al_ref[:, h:h + 1] = lax.dot_general(
            hh, asrc_ref[h:h + 1, :], (((1,), (1,)), ((), ())),
            preferred_element_type=jnp.float32)
        al_ref[:, H + h:H + h + 1] = lax.dot_general(
            hh, adst_ref[h:h + 1, :], (((1,), (1,)), ((), ())),
            preferred_element_type=jnp.float32)


def _hidden_al(x, gw, gas, gad):
    bn = 1000
    return pl.pallas_call(
        _hal_body,
        grid=(N // bn,),
        in_specs=[pl.BlockSpec((bn, D), lambda i: (i, 0)),
                  pl.BlockSpec((H, D, D), lambda i: (0, 0, 0)),
                  pl.BlockSpec((H, D), lambda i: (0, 0)),
                  pl.BlockSpec((H, D), lambda i: (0, 0))],
        out_specs=[pl.BlockSpec((bn, H * D), lambda i: (i, 0)),
                   pl.BlockSpec((bn, 2 * H), lambda i: (i, 0))],
        out_shape=[jax.ShapeDtypeStruct((N, H * D), jnp.float32),
                   jax.ShapeDtypeStruct((N, 2 * H), jnp.float32)],
    )(x, gw, gas, gad)


def _sc_partition(src, dst):
    mesh = plsc.VectorSubcoreMesh(core_axis_name="c", subcore_axis_name="s")

    @functools.partial(
        pl.kernel,
        mesh=mesh,
        out_type=(jax.ShapeDtypeStruct((NTILES * NTILES * CAPP,), jnp.int32),
                  jax.ShapeDtypeStruct((NTILES * NTILES * CAPP,), jnp.int32),
                  jax.ShapeDtypeStruct((NTILES * NTILES,), jnp.int32)),
        compiler_params=pltpu.CompilerParams(needs_layout_passes=False,
                                             use_tc_tiling_on_sc=False),
        scratch_types=[
            pltpu.VMEM((EPPT,), jnp.int32),
            pltpu.VMEM((EPPT,), jnp.int32),
            pltpu.VMEM((LANES,), jnp.int32),
            pltpu.VMEM((LANES,), jnp.int32),
            pltpu.VMEM((NTILES * 32,), jnp.int32),
            pltpu.VMEM((NTILES * 32,), jnp.int32),
            pltpu.VMEM((NTILES,), jnp.int32),
            pltpu.VMEM((NTILES,), jnp.int32),
            pltpu.SemaphoreType.DMA,
        ],
    )
    def k(src_hbm, dst_hbm, bsrc_hbm, bdst_hbm, cnts_hbm,
          srcb, dstb, xsrc, xdst, stgs, stgd, scnt, wrt, semi):
        cid = lax.axis_index("c")
        sid = lax.axis_index("s")
        wid = cid * 16 + sid
        iota = lax.iota(jnp.int32, LANES)
        zi = jnp.zeros((LANES,), jnp.int32)

        for gz in range(NTILES * 32 // LANES):
            stgs[pl.ds(gz * LANES, LANES)] = zi
            stgd[pl.ds(gz * LANES, LANES)] = zi
        scnt[pl.ds(0, LANES)] = zi
        scnt[pl.ds(LANES, LANES)] = zi
        wrt[pl.ds(0, LANES)] = zi
        wrt[pl.ds(LANES, LANES)] = zi

        def do_group(sidx, didx):
            bv = lax.shift_right_logical(didx * DIVM, DIVS)
            for b in range(NTILES):
                grp = (b // LANES) * LANES
                ln = b % LANES
                m = bv == b
                c = plsc.all_reduce_population_count(m)[0]
                scv = scnt[pl.ds(grp, LANES)]
                sc_b = scv[ln]
                plsc.store_compressed(stgs.at[pl.ds(b * 32 + sc_b, LANES)],
                                      sidx, mask=m)
                plsc.store_compressed(stgd.at[pl.ds(b * 32 + sc_b, LANES)],
                                      didx, mask=m)
                nc = sc_b + c

                @pl.when(nc >= LANES)
                def _():
                    wv = wrt[pl.ds(grp, LANES)]
                    w_b = wv[ln]
                    base = pl.multiple_of((b * NTILES + wid) * CAPP + w_b, 8)
                    pltpu.sync_copy(stgs.at[pl.ds(b * 32, LANES)],
                                    bsrc_hbm.at[pl.ds(base, LANES)])
                    pltpu.sync_copy(stgd.at[pl.ds(b * 32, LANES)],
                                    bdst_hbm.at[pl.ds(base, LANES)])
                    stgs[pl.ds(b * 32, LANES)] = stgs[pl.ds(b * 32 + LANES,
                                                            LANES)]
                    stgd[pl.ds(b * 32, LANES)] = stgd[pl.ds(b * 32 + LANES,
                                                            LANES)]
                    wrt[pl.ds(grp, LANES)] = jnp.where(iota == ln,
                                                       wv + LANES, wv)
                nc2 = jnp.where(nc >= LANES, nc - LANES, nc)
                scv2 = scnt[pl.ds(grp, LANES)]
                scnt[pl.ds(grp, LANES)] = jnp.where(iota == ln, nc2, scv2)

        ls = pltpu.async_copy(src_hbm.at[pl.ds(pl.multiple_of(wid * EPPT, 8), EPPT)], srcb, semi)
        ld = pltpu.async_copy(dst_hbm.at[pl.ds(pl.multiple_of(wid * EPPT, 8), EPPT)], dstb, semi)
        ls.wait()
        ld.wait()

        def group_loop(g, _):
            sl = pl.ds(g * LANES, LANES)
            do_group(srcb[sl], dstb[sl])
            return 0
        lax.fori_loop(0, EPPT // LANES, group_loop, 0)

        @pl.when(wid < (E - NTILES * EPPT) // LANES)
        def _():
            xs = pltpu.async_copy(
                src_hbm.at[pl.ds(pl.multiple_of(NTILES * EPPT + wid * LANES, 8), LANES)],
                xsrc, semi)
            xd = pltpu.async_copy(
                dst_hbm.at[pl.ds(pl.multiple_of(NTILES * EPPT + wid * LANES, 8), LANES)],
                xdst, semi)
            xs.wait()
            xd.wait()
            do_group(xsrc[pl.ds(0, LANES)], xdst[pl.ds(0, LANES)])

        for b in range(NTILES):
            grp = (b // LANES) * LANES
            ln = b % LANES
            scv = scnt[pl.ds(grp, LANES)]
            sc_b = scv[ln]

            @pl.when(sc_b > 0)
            def _():
                wv = wrt[pl.ds(grp, LANES)]
                w_b = wv[ln]
                base = pl.multiple_of((b * NTILES + wid) * CAPP + w_b, 8)
                pltpu.sync_copy(stgs.at[pl.ds(b * 32, LANES)],
                                bsrc_hbm.at[pl.ds(base, LANES)])
                pltpu.sync_copy(stgd.at[pl.ds(b * 32, LANES)],
                                bdst_hbm.at[pl.ds(base, LANES)])
        c0 = wrt[pl.ds(0, LANES)] + scnt[pl.ds(0, LANES)]
        c1 = wrt[pl.ds(LANES, LANES)] + scnt[pl.ds(LANES, LANES)]
        scnt[pl.ds(0, LANES)] = c0
        scnt[pl.ds(LANES, LANES)] = c1
        pltpu.sync_copy(scnt, cnts_hbm.at[pl.ds(pl.multiple_of(wid * NTILES, 8), NTILES)])

    return k(src, dst)


def _sc_edge_agg(hcat, als0, ald0, als1, ald1, bsrc, bdst, cnts):
    mesh = plsc.VectorSubcoreMesh(core_axis_name="c", subcore_axis_name="s")

    @functools.partial(
        pl.kernel,
        mesh=mesh,
        out_type=(jax.ShapeDtypeStruct((H, ROWS, D), jnp.float32),
                  jax.ShapeDtypeStruct((ROWS, LANES), jnp.float32)),
        compiler_params=pltpu.CompilerParams(needs_layout_passes=False,
                                             use_tc_tiling_on_sc=False),
        scratch_types=[
            pltpu.VMEM((NPB, D), jnp.float32),
            pltpu.VMEM((NPB, D), jnp.float32),
            pltpu.VMEM((NPB, LANES), jnp.float32),
            pltpu.VMEM((NTILES * NTILES,), jnp.int32),
            pltpu.VMEM((PROC,), jnp.int32),
            pltpu.VMEM((PROC,), jnp.int32),
            pltpu.VMEM((PROC,), jnp.int32),
            pltpu.VMEM((PROC,), jnp.float32),
            pltpu.VMEM((PROC,), jnp.float32),
            pltpu.VMEM((PROC,), jnp.float32),
            pltpu.VMEM((PROC,), jnp.float32),
            pltpu.VMEM((PROC,), jnp.float32),
            pltpu.VMEM((PROC,), jnp.float32),
            pltpu.VMEM((PROC, H * D), jnp.float32),
            pltpu.SemaphoreType.DMA,
            pltpu.SemaphoreType.DMA,
            pltpu.SemaphoreType.DMA,
        ],
    )
    def k(h_hbm, als0_hbm, ald0_hbm, als1_hbm, ald1_hbm,
          bsrc_hbm, bdst_hbm, cnts_hbm,
          pay_hbm, wsum_hbm,
          acc0, acc1, accw, cntv, ssrc, sdst, dlb,
          w0b, w1b, av0, ad0, av1, ad1, rowsb, semr, sema, semi):
        cid = lax.axis_index("c")
        sid = lax.axis_index("s")
        wid = cid * 16 + sid
        lo = wid * NPB
        iota = lax.iota(jnp.int32, LANES)
        zf = jnp.zeros((LANES,), jnp.float32)
        e0 = (iota == 0).astype(jnp.float32)
        e1 = (iota == 1).astype(jnp.float32)

        cg = pltpu.async_copy(cnts_hbm, cntv, semi)

        def zacc(i, _):
            for dpart in range(D // LANES):
                acc0[i, pl.ds(dpart * LANES, LANES)] = zf
                acc1[i, pl.ds(dpart * LANES, LANES)] = zf
            accw[i, :] = zf
            return 0
        lax.fori_loop(0, NPB, zacc, 0)
        cg.wait()

        def process_batch(count):
            gr = pltpu.async_copy(h_hbm.at[ssrc], rowsb, semr)
            g0 = pltpu.async_copy(als0_hbm.at[ssrc], av0, sema)
            g1 = pltpu.async_copy(ald0_hbm.at[sdst], ad0, sema)
            g2 = pltpu.async_copy(als1_hbm.at[ssrc], av1, sema)
            g3 = pltpu.async_copy(ald1_hbm.at[sdst], ad1, sema)
            g0.wait()
            g1.wait()
            g2.wait()
            g3.wait()
            for g in range(PROC // LANES):
                sl = pl.ds(g * LANES, LANES)
                valid = (g * LANES + iota) < count
                x0 = av0[sl] + ad0[sl]
                w0 = jnp.where(valid, jnp.exp(jnp.maximum(x0, 0.2 * x0)), 0.0)
                x1 = av1[sl] + ad1[sl]
                w1 = jnp.where(valid, jnp.exp(jnp.maximum(x1, 0.2 * x1)), 0.0)
                w0b[sl] = w0
                w1b[sl] = w1
                dl = sdst[sl] - lo
                dlb[sl] = jnp.minimum(jnp.maximum(dl, 0), NPB - 1)
            gr.wait()

            def accum(g, _):
                sl = pl.ds(g * LANES, LANES)
                dl16 = dlb[sl]
                w016 = w0b[sl]
                w116 = w1b[sl]
                for j in range(LANES):
                    i = g * LANES + j
                    dlj = dl16[j]
                    w0j = w016[j]
                    w1j = w116[j]
                    for dpart in range(D // LANES):
                        c = pl.ds(dpart * LANES, LANES)
                        r0 = rowsb[i, c]
                        acc0[dlj, c] = acc0[dlj, c] + w0j * r0
                        r1 = rowsb[i, pl.ds(D + dpart * LANES, LANES)]
                        acc1[dlj, c] = acc1[dlj, c] + w1j * r1
                    accw[dlj, :] = accw[dlj, :] + w0j * e0 + w1j * e1
                return 0
            lax.fori_loop(0, PROC // LANES, accum, 0)

        def per_p(p, _):
            ci = plsc.load_gather(
                cntv, [jnp.zeros((LANES,), jnp.int32) + (p * NTILES + wid)])
            cnt = ci[0]
            base = (wid * NTILES + p) * CAPP
            nb = (cnt + PROC - 1) // PROC

            def per_batch(kb, _):
                off = pl.multiple_of(base + kb * PROC, 8)
                ls = pltpu.async_copy(bsrc_hbm.at[pl.ds(off, PROC)],
                                      ssrc, semi)
                ld = pltpu.async_copy(bdst_hbm.at[pl.ds(off, PROC)],
                                      sdst, semi)
                ls.wait()
                ld.wait()
                for g in range(PROC // LANES):
                    sl = pl.ds(g * LANES, LANES)
                    ssrc[sl] = jnp.minimum(jnp.maximum(ssrc[sl], 0), N - 1)
                    sdst[sl] = jnp.minimum(jnp.maximum(sdst[sl], 0), N - 1)
                process_batch(cnt - kb * PROC)
                return 0
            lax.fori_loop(0, nb, per_batch, 0)
            return 0
        lax.fori_loop(0, NTILES, per_p, 0)

        pltpu.sync_copy(acc0, pay_hbm.at[0, pl.ds(lo, NPB)])
        pltpu.sync_copy(acc1, pay_hbm.at[1, pl.ds(lo, NPB)])
        pltpu.sync_copy(accw, wsum_hbm.at[pl.ds(lo, NPB)])

    return k(hcat, als0, ald0, als1, ald1, bsrc, bdst, cnts)


def _comb_body(p_ref, w_ref, o_ref):
    accm = None
    for h in range(H):
        num = p_ref[h]
        den = w_ref[:, h:h + 1]
        v = num / (den + 1e-16)
        e = jnp.where(v > 0, v, jnp.exp(jnp.minimum(v, 0.0)) - 1.0)
        accm = e if accm is None else accm + e
    m = accm * (1.0 / H)
    nrm = jnp.sqrt(jnp.sum(m * m, axis=1, keepdims=True))
    o_ref[...] = m / (nrm + 1e-12)


def _combine(pay, wsum):
    bn = 632
    return pl.pallas_call(
        _comb_body,
        grid=(ROWS // bn,),
        in_specs=[pl.BlockSpec((H, bn, D), lambda i: (0, i, 0)),
                  pl.BlockSpec((bn, LANES), lambda i: (i, 0))],
        out_specs=pl.BlockSpec((bn, D), lambda i: (i, 0)),
        out_shape=jax.ShapeDtypeStruct((ROWS, D), jnp.float32),
    )(pay, wsum)


_INV_SQRT_D = 0.08838834764831845


def _mha_body(x0_ref, x1_ref, x2_ref, wq_ref, wk_ref, wv_ref, o_ref):
    xs = [x0_ref[...], x1_ref[...], x2_ref[...]]
    for h in range(H):
        q = [jnp.dot(x, wq_ref[h], preferred_element_type=jnp.float32)
             for x in xs]
        kk = [jnp.dot(x, wk_ref[h], preferred_element_type=jnp.float32)
              for x in xs]
        vv = [jnp.dot(x, wv_ref[h], preferred_element_type=jnp.float32)
              for x in xs]
        osum = None
        for r in range(R):
            att = [jnp.sum(q[r] * kk[s], axis=1, keepdims=True) * _INV_SQRT_D
                   for s in range(R)]
            m = jnp.maximum(jnp.maximum(att[0], att[1]), att[2])
            ee = [jnp.exp(a - m) for a in att]
            den = ee[0] + ee[1] + ee[2]
            o_r = (ee[0] * vv[0] + ee[1] * vv[1] + ee[2] * vv[2]) / den
            osum = o_r if osum is None else osum + o_r
        o_ref[:, h * D:(h + 1) * D] = osum * (1.0 / R)


def _mha(x0, x1, x2, wq, wk, wv):
    bn = 1000
    return pl.pallas_call(
        _mha_body,
        grid=(N // bn,),
        in_specs=[pl.BlockSpec((bn, D), lambda i: (i, 0)),
                  pl.BlockSpec((bn, D), lambda i: (i, 0)),
                  pl.BlockSpec((bn, D), lambda i: (i, 0)),
                  pl.BlockSpec((H, D, D), lambda i: (0, 0, 0)),
                  pl.BlockSpec((H, D, D), lambda i: (0, 0, 0)),
                  pl.BlockSpec((H, D, D), lambda i: (0, 0, 0))],
        out_specs=pl.BlockSpec((bn, H * D), lambda i: (i, 0)),
        out_shape=jax.ShapeDtypeStruct((N, H * D), jnp.float32),
    )(x0, x1, x2, wq, wk, wv)


def _proj_body(adj_ref, emb_ref, fused_ref, w_ref, b_ref, o_ref):
    adj = adj_ref[...]
    rs = jnp.sum(adj, axis=1, keepdims=True)
    ragg = jnp.dot(adj, emb_ref[...],
                   preferred_element_type=jnp.float32) / (rs + 1e-5)
    f = jnp.dot(fused_ref[...], w_ref[:H * D, :],
                preferred_element_type=jnp.float32)
    g = jnp.dot(ragg, w_ref[H * D:, :], preferred_element_type=jnp.float32)
    o_ref[...] = jnp.maximum(f + g + b_ref[...], 0.0)


def _relproj(rel_adj, rel_emb, fused, proj_w, proj_b2):
    bn = 1000
    return pl.pallas_call(
        _proj_body,
        grid=(N // bn,),
        in_specs=[pl.BlockSpec((bn, RN), lambda i: (i, 0)),
                  pl.BlockSpec((RN, D), lambda i: (0, 0)),
                  pl.BlockSpec((bn, H * D), lambda i: (i, 0)),
                  pl.BlockSpec((H * D + D, D), lambda i: (0, 0)),
                  pl.BlockSpec((1, D), lambda i: (0, 0))],
        out_specs=pl.BlockSpec((bn, D), lambda i: (i, 0)),
        out_shape=jax.ShapeDtypeStruct((N, D), jnp.float32),
    )(rel_adj, rel_emb, fused, proj_w, proj_b2)


def _forward(ent, rel_emb, rel_adj, edge, gat_w, gat_asrc, gat_adst,
             wq, wk, wv, proj_w, proj_b2):
    srcp = edge[0].astype(jnp.int32)
    dstp = edge[1].astype(jnp.int32)
    bsrc, bdst, cnts = _sc_partition(srcp, dstp)
    xs = [ent]
    x = ent
    for l in range(NLAYERS):
        hcat, al = _hidden_al(x, gat_w[l], gat_asrc[l], gat_adst[l])
        pay, wsum = _sc_edge_agg(hcat, al[:, 0], al[:, 2],
                                 al[:, 1], al[:, 3], bsrc, bdst, cnts)
        x = _combine(pay, wsum)[:N]
        xs.append(x)
    fused = _mha(xs[0], xs[1], xs[2], wq, wk, wv)
    return _relproj(rel_adj, rel_emb, fused, proj_w, proj_b2)


def kernel(ent_sr, ent_tg, rel_emb_sr, rel_emb_tg, rel_adj_sr, rel_adj_tg,
           gat_W, gat_asrc, gat_adst, Wq, Wk, Wv, proj_W, proj_b,
           edge_sr, edge_tg):
    pb = proj_b.reshape(1, D)
    sr = _forward(ent_sr, rel_emb_sr, rel_adj_sr, edge_sr,
                  gat_W, gat_asrc, gat_adst, Wq, Wk, Wv, proj_W, pb)
    tg = _forward(ent_tg, rel_emb_tg, rel_adj_tg, edge_tg,
                  gat_W, gat_asrc, gat_adst, Wq, Wk, Wv, proj_W, pb)
    return (sr, tg)


# speedup vs baseline: 1.2625x; 1.2625x over previous
import functools

import jax
import jax.numpy as jnp
from jax import lax
from jax.experimental import pallas as pl
from jax.experimental.pallas import tpu as pltpu
from jax.experimental.pallas import tpu_sc as plsc

N = 10000
D = 128
H = 2
E = 160000
RN = 1000
R = 3
NLAYERS = 2

LANES = 16
NTILES = 32
ROWS = 10112
NPB = ROWS // NTILES
PROC = 96
CAPP = 5120
EPPT = 4992
DIVM = 26547
DIVS = 23


HW = 272


def _hal_body(x_ref, w_ref, asrc_ref, adst_ref, h_ref, al_ref):
    x = x_ref[...]
    for h in range(H):
        hh = jnp.dot(x, w_ref[h], preferred_element_type=jnp.float32)
        h_ref[:, h * D:(h + 1) * D] = hh
        h_ref[:, H * D + h:H * D + h + 1] = lax.dot_general(
            hh, asrc_ref[h:h + 1, :], (((1,), (1,)), ((), ())),
            preferred_element_type=jnp.float32)
        al_ref[:, h:h + 1] = lax.dot_general(
            hh, adst_ref[h:h + 1, :], (((1,), (1,)), ((), ())),
            preferred_element_type=jnp.float32)


def _hidden_al(x, gw, gas, gad):
    bn = 1000
    return pl.pallas_call(
        _hal_body,
        grid=(N // bn,),
        in_specs=[pl.BlockSpec((bn, D), lambda i: (i, 0)),
                  pl.BlockSpec((H, D, D), lambda i: (0, 0, 0)),
                  pl.BlockSpec((H, D), lambda i: (0, 0)),
                  pl.BlockSpec((H, D), lambda i: (0, 0))],
        out_specs=[pl.BlockSpec((bn, HW), lambda i: (i, 0)),
                   pl.BlockSpec((bn, LANES), lambda i: (i, 0))],
        out_shape=[jax.ShapeDtypeStruct((N, HW), jnp.float32),
                   jax.ShapeDtypeStruct((N, LANES), jnp.float32)],
    )(x, gw, gas, gad)


def _sc_partition(src, dst):
    mesh = plsc.VectorSubcoreMesh(core_axis_name="c", subcore_axis_name="s")

    @functools.partial(
        pl.kernel,
        mesh=mesh,
        out_type=(jax.ShapeDtypeStruct((NTILES * NTILES * CAPP,), jnp.int32),
                  jax.ShapeDtypeStruct((NTILES * NTILES * CAPP,), jnp.int32),
                  jax.ShapeDtypeStruct((NTILES * NTILES,), jnp.int32)),
        compiler_params=pltpu.CompilerParams(needs_layout_passes=False,
                                             use_tc_tiling_on_sc=False),
        scratch_types=[
            pltpu.VMEM((EPPT,), jnp.int32),
            pltpu.VMEM((EPPT,), jnp.int32),
            pltpu.VMEM((LANES,), jnp.int32),
            pltpu.VMEM((LANES,), jnp.int32),
            pltpu.VMEM((NTILES * 64,), jnp.int32),
            pltpu.VMEM((NTILES * 64,), jnp.int32),
            pltpu.VMEM((NTILES,), jnp.int32),
            pltpu.VMEM((NTILES,), jnp.int32),
            pltpu.SemaphoreType.DMA,
        ],
    )
    def k(src_hbm, dst_hbm, bsrc_hbm, bdst_hbm, cnts_hbm,
          srcb, dstb, xsrc, xdst, stgs, stgd, scnt, wrt, semi):
        cid = lax.axis_index("c")
        sid = lax.axis_index("s")
        wid = cid * 16 + sid
        iota = lax.iota(jnp.int32, LANES)
        zi = jnp.zeros((LANES,), jnp.int32)

        for gz in range(NTILES * 64 // LANES):
            stgs[pl.ds(gz * LANES, LANES)] = zi
            stgd[pl.ds(gz * LANES, LANES)] = zi
        scnt[pl.ds(0, LANES)] = zi
        scnt[pl.ds(LANES, LANES)] = zi
        wrt[pl.ds(0, LANES)] = zi
        wrt[pl.ds(LANES, LANES)] = zi

        def do_group(sidx, didx):
            bv = lax.shift_right_logical(didx * DIVM, DIVS)
            for b in range(NTILES):
                grp = (b // LANES) * LANES
                ln = b % LANES
                m = bv == b
                c = plsc.all_reduce_population_count(m)[0]
                scv = scnt[pl.ds(grp, LANES)]
                sc_b = scv[ln]
                plsc.store_compressed(stgs.at[pl.ds(b * 64 + sc_b, LANES)],
                                      sidx, mask=m)
                plsc.store_compressed(stgd.at[pl.ds(b * 64 + sc_b, LANES)],
                                      didx, mask=m)
                nc = sc_b + c

                @pl.when(nc >= 48)
                def _():
                    wv = wrt[pl.ds(grp, LANES)]
                    w_b = wv[ln]
                    base = pl.multiple_of((b * NTILES + wid) * CAPP + w_b, 8)
                    pltpu.sync_copy(stgs.at[pl.ds(b * 64, 48)],
                                    bsrc_hbm.at[pl.ds(base, 48)])
                    pltpu.sync_copy(stgd.at[pl.ds(b * 64, 48)],
                                    bdst_hbm.at[pl.ds(base, 48)])
                    stgs[pl.ds(b * 64, LANES)] = stgs[pl.ds(b * 64 + 48,
                                                            LANES)]
                    stgd[pl.ds(b * 64, LANES)] = stgd[pl.ds(b * 64 + 48,
                                                            LANES)]
                    wrt[pl.ds(grp, LANES)] = jnp.where(iota == ln,
                                                       wv + 48, wv)
                nc2 = jnp.where(nc >= 48, nc - 48, nc)
                scv2 = scnt[pl.ds(grp, LANES)]
                scnt[pl.ds(grp, LANES)] = jnp.where(iota == ln, nc2, scv2)

        ls = pltpu.async_copy(src_hbm.at[pl.ds(pl.multiple_of(wid * EPPT, 8), EPPT)], srcb, semi)
        ld = pltpu.async_copy(dst_hbm.at[pl.ds(pl.multiple_of(wid * EPPT, 8), EPPT)], dstb, semi)
        ls.wait()
        ld.wait()

        def group_loop(g, _):
            sl = pl.ds(g * LANES, LANES)
            do_group(srcb[sl], dstb[sl])
            return 0
        lax.fori_loop(0, EPPT // LANES, group_loop, 0)

        @pl.when(wid < (E - NTILES * EPPT) // LANES)
        def _():
            xs = pltpu.async_copy(
                src_hbm.at[pl.ds(pl.multiple_of(NTILES * EPPT + wid * LANES, 8), LANES)],
                xsrc, semi)
            xd = pltpu.async_copy(
                dst_hbm.at[pl.ds(pl.multiple_of(NTILES * EPPT + wid * LANES, 8), LANES)],
                xdst, semi)
            xs.wait()
            xd.wait()
            do_group(xsrc[pl.ds(0, LANES)], xdst[pl.ds(0, LANES)])

        for b in range(NTILES):
            grp = (b // LANES) * LANES
            ln = b % LANES
            scv = scnt[pl.ds(grp, LANES)]
            sc_b = scv[ln]

            @pl.when(sc_b > 0)
            def _():
                wv = wrt[pl.ds(grp, LANES)]
                w_b = wv[ln]
                base = pl.multiple_of((b * NTILES + wid) * CAPP + w_b, 8)
                pltpu.sync_copy(stgs.at[pl.ds(b * 64, 48)],
                                bsrc_hbm.at[pl.ds(base, 48)])
                pltpu.sync_copy(stgd.at[pl.ds(b * 64, 48)],
                                bdst_hbm.at[pl.ds(base, 48)])
        c0 = wrt[pl.ds(0, LANES)] + scnt[pl.ds(0, LANES)]
        c1 = wrt[pl.ds(LANES, LANES)] + scnt[pl.ds(LANES, LANES)]
        scnt[pl.ds(0, LANES)] = c0
        scnt[pl.ds(LANES, LANES)] = c1
        pltpu.sync_copy(scnt, cnts_hbm.at[pl.ds(pl.multiple_of(wid * NTILES, 8), NTILES)])

    return k(src, dst)


def _sc_edge_agg(hcat, aldt, bsrc, bdst, cnts):
    mesh = plsc.VectorSubcoreMesh(core_axis_name="c", subcore_axis_name="s")

    @functools.partial(
        pl.kernel,
        mesh=mesh,
        out_type=(jax.ShapeDtypeStruct((H, ROWS, D), jnp.float32),
                  jax.ShapeDtypeStruct((ROWS, LANES), jnp.float32)),
        compiler_params=pltpu.CompilerParams(needs_layout_passes=False,
                                             use_tc_tiling_on_sc=False),
        scratch_types=[
            pltpu.VMEM((NPB, D), jnp.float32),
            pltpu.VMEM((NPB, D), jnp.float32),
            pltpu.VMEM((NPB, LANES), jnp.float32),
            pltpu.VMEM((NPB, LANES), jnp.float32),
            pltpu.VMEM((NTILES * NTILES,), jnp.int32),
            pltpu.VMEM((2 * PROC,), jnp.int32),
            pltpu.VMEM((2 * PROC,), jnp.int32),
            pltpu.VMEM((PROC,), jnp.int32),
            pltpu.VMEM((PROC,), jnp.int32),
            pltpu.VMEM((PROC,), jnp.int32),
            pltpu.VMEM((PROC,), jnp.float32),
            pltpu.VMEM((PROC,), jnp.float32),
            pltpu.VMEM((PROC, HW), jnp.float32),
            pltpu.SemaphoreType.DMA,
            pltpu.SemaphoreType.DMA,
        ],
    )
    def k(h_hbm, aldt_hbm, bsrc_hbm, bdst_hbm, cnts_hbm,
          pay_hbm, wsum_hbm,
          acc0, acc1, accw, albuf, cntv, ps, pd, ssrc, sdst, dlb,
          w0b, w1b, rowsb, semr, semi):
        cid = lax.axis_index("c")
        sid = lax.axis_index("s")
        wid = cid * 16 + sid
        lo = wid * NPB
        iota = lax.iota(jnp.int32, LANES)
        zf = jnp.zeros((LANES,), jnp.float32)
        e0 = (iota == 0).astype(jnp.float32)
        e1 = (iota == 1).astype(jnp.float32)
        c256 = jnp.zeros((LANES,), jnp.int32) + (H * D)
        c257 = jnp.zeros((LANES,), jnp.int32) + (H * D + 1)
        cz = jnp.zeros((LANES,), jnp.int32)
        co = jnp.zeros((LANES,), jnp.int32) + 1

        cg = pltpu.async_copy(cnts_hbm, cntv, semi)
        ag = pltpu.async_copy(aldt_hbm.at[pl.ds(lo, NPB)], albuf, semr)

        def zacc(i, _):
            for dpart in range(D // LANES):
                acc0[i, pl.ds(dpart * LANES, LANES)] = zf
                acc1[i, pl.ds(dpart * LANES, LANES)] = zf
            accw[i, :] = zf
            return 0
        lax.fori_loop(0, NPB, zacc, 0)
        cg.wait()
        ag.wait()

        def process_batch(count):
            gr = pltpu.async_copy(h_hbm.at[ssrc], rowsb, semr)
            for g in range(PROC // LANES):
                sl = pl.ds(g * LANES, LANES)
                dl = sdst[sl] - lo
                dlb[sl] = jnp.minimum(jnp.maximum(dl, 0), NPB - 1)
            gr.wait()
            for g in range(PROC // LANES):
                sl = pl.ds(g * LANES, LANES)
                ei = g * LANES + iota
                valid = ei < count
                dl16 = dlb[sl]
                a0 = plsc.load_gather(rowsb, [ei, c256])
                a1 = plsc.load_gather(rowsb, [ei, c257])
                b0 = plsc.load_gather(albuf, [dl16, cz])
                b1 = plsc.load_gather(albuf, [dl16, co])
                x0 = a0 + b0
                w0b[sl] = jnp.where(valid,
                                    jnp.exp(jnp.maximum(x0, 0.2 * x0)), 0.0)
                x1 = a1 + b1
                w1b[sl] = jnp.where(valid,
                                    jnp.exp(jnp.maximum(x1, 0.2 * x1)), 0.0)

            def accum(g, _):
                sl = pl.ds(g * LANES, LANES)
                dl16 = dlb[sl]
                w016 = w0b[sl]
                w116 = w1b[sl]
                for j in range(LANES):
                    i = g * LANES + j
                    dlj = dl16[j]
                    w0j = w016[j]
                    w1j = w116[j]
                    for dpart in range(D // LANES):
                        c = pl.ds(dpart * LANES, LANES)
                        r0 = rowsb[i, c]
                        acc0[dlj, c] = acc0[dlj, c] + w0j * r0
                        r1 = rowsb[i, pl.ds(D + dpart * LANES, LANES)]
                        acc1[dlj, c] = acc1[dlj, c] + w1j * r1
                    accw[dlj, :] = accw[dlj, :] + w0j * e0 + w1j * e1
                return 0
            lax.fori_loop(0, PROC // LANES, accum, 0)

        def per_p(p, _):
            ci = plsc.load_gather(
                cntv, [jnp.zeros((LANES,), jnp.int32) + (p * NTILES + wid)])
            cnt = ci[0]
            base = (wid * NTILES + p) * CAPP
            nb = (cnt + PROC - 1) // PROC
            lp = pltpu.async_copy(
                bsrc_hbm.at[pl.ds(pl.multiple_of(base, 8), 2 * PROC)],
                ps, semi)
            ld = pltpu.async_copy(
                bdst_hbm.at[pl.ds(pl.multiple_of(base, 8), 2 * PROC)],
                pd, semi)
            lp.wait()
            ld.wait()

            def per_batch(kb, _):
                @pl.when(kb < 2)
                def _():
                    for g in range(PROC // LANES):
                        sl = pl.ds(g * LANES, LANES)
                        psl = pl.ds(kb * PROC + g * LANES, LANES)
                        ssrc[sl] = jnp.minimum(jnp.maximum(ps[psl], 0), N - 1)
                        sdst[sl] = jnp.minimum(jnp.maximum(pd[psl], 0), N - 1)

                @pl.when(kb >= 2)
                def _():
                    off = pl.multiple_of(base + kb * PROC, 8)
                    ls2 = pltpu.async_copy(bsrc_hbm.at[pl.ds(off, PROC)],
                                           ssrc, semi)
                    ld2 = pltpu.async_copy(bdst_hbm.at[pl.ds(off, PROC)],
                                           sdst, semi)
                    ls2.wait()
                    ld2.wait()
                    for g in range(PROC // LANES):
                        sl = pl.ds(g * LANES, LANES)
                        ssrc[sl] = jnp.minimum(jnp.maximum(ssrc[sl], 0), N - 1)
                        sdst[sl] = jnp.minimum(jnp.maximum(sdst[sl], 0), N - 1)
                process_batch(cnt - kb * PROC)
                return 0
            lax.fori_loop(0, nb, per_batch, 0)
            return 0
        lax.fori_loop(0, NTILES, per_p, 0)

        pltpu.sync_copy(acc0, pay_hbm.at[0, pl.ds(lo, NPB)])
        pltpu.sync_copy(acc1, pay_hbm.at[1, pl.ds(lo, NPB)])
        pltpu.sync_copy(accw, wsum_hbm.at[pl.ds(lo, NPB)])

    return k(hcat, aldt, bsrc, bdst, cnts)


def _comb_body(p_ref, w_ref, o_ref):
    accm = None
    for h in range(H):
        num = p_ref[h]
        den = w_ref[:, h:h + 1]
        v = num / (den + 1e-16)
        e = jnp.where(v > 0, v, jnp.exp(jnp.minimum(v, 0.0)) - 1.0)
        accm = e if accm is None else accm + e
    m = accm * (1.0 / H)
    nrm = jnp.sqrt(jnp.sum(m * m, axis=1, keepdims=True))
    o_ref[...] = m / (nrm + 1e-12)


def _combine(pay, wsum):
    bn = 632
    return pl.pallas_call(
        _comb_body,
        grid=(ROWS // bn,),
        in_specs=[pl.BlockSpec((H, bn, D), lambda i: (0, i, 0)),
                  pl.BlockSpec((bn, LANES), lambda i: (i, 0))],
        out_specs=pl.BlockSpec((bn, D), lambda i: (i, 0)),
        out_shape=jax.ShapeDtypeStruct((ROWS, D), jnp.float32),
    )(pay, wsum)


_INV_SQRT_D = 0.08838834764831845


def _mha_body(x0_ref, x1_ref, x2_ref, wq_ref, wk_ref, wv_ref, o_ref):
    xs = [x0_ref[...], x1_ref[...], x2_ref[...]]
    for h in range(H):
        q = [jnp.dot(x, wq_ref[h], preferred_element_type=jnp.float32)
             for x in xs]
        kk = [jnp.dot(x, wk_ref[h], preferred_element_type=jnp.float32)
              for x in xs]
        vv = [jnp.dot(x, wv_ref[h], preferred_element_type=jnp.float32)
              for x in xs]
        osum = None
        for r in range(R):
            att = [jnp.sum(q[r] * kk[s], axis=1, keepdims=True) * _INV_SQRT_D
                   for s in range(R)]
            m = jnp.maximum(jnp.maximum(att[0], att[1]), att[2])
            ee = [jnp.exp(a - m) for a in att]
            den = ee[0] + ee[1] + ee[2]
            o_r = (ee[0] * vv[0] + ee[1] * vv[1] + ee[2] * vv[2]) / den
            osum = o_r if osum is None else osum + o_r
        o_ref[:, h * D:(h + 1) * D] = osum * (1.0 / R)


def _mha(x0, x1, x2, wq, wk, wv):
    bn = 1000
    return pl.pallas_call(
        _mha_body,
        grid=(N // bn,),
        in_specs=[pl.BlockSpec((bn, D), lambda i: (i, 0)),
                  pl.BlockSpec((bn, D), lambda i: (i, 0)),
                  pl.BlockSpec((bn, D), lambda i: (i, 0)),
                  pl.BlockSpec((H, D, D), lambda i: (0, 0, 0)),
                  pl.BlockSpec((H, D, D), lambda i: (0, 0, 0)),
                  pl.BlockSpec((H, D, D), lambda i: (0, 0, 0))],
        out_specs=pl.BlockSpec((bn, H * D), lambda i: (i, 0)),
        out_shape=jax.ShapeDtypeStruct((N, H * D), jnp.float32),
    )(x0, x1, x2, wq, wk, wv)


def _proj_body(adj_ref, emb_ref, fused_ref, w_ref, b_ref, o_ref):
    adj = adj_ref[...]
    rs = jnp.sum(adj, axis=1, keepdims=True)
    ragg = jnp.dot(adj, emb_ref[...],
                   preferred_element_type=jnp.float32) / (rs + 1e-5)
    f = jnp.dot(fused_ref[...], w_ref[:H * D, :],
                preferred_element_type=jnp.float32)
    g = jnp.dot(ragg, w_ref[H * D:, :], preferred_element_type=jnp.float32)
    o_ref[...] = jnp.maximum(f + g + b_ref[...], 0.0)


def _relproj(rel_adj, rel_emb, fused, proj_w, proj_b2):
    bn = 1000
    return pl.pallas_call(
        _proj_body,
        grid=(N // bn,),
        in_specs=[pl.BlockSpec((bn, RN), lambda i: (i, 0)),
                  pl.BlockSpec((RN, D), lambda i: (0, 0)),
                  pl.BlockSpec((bn, H * D), lambda i: (i, 0)),
                  pl.BlockSpec((H * D + D, D), lambda i: (0, 0)),
                  pl.BlockSpec((1, D), lambda i: (0, 0))],
        out_specs=pl.BlockSpec((bn, D), lambda i: (i, 0)),
        out_shape=jax.ShapeDtypeStruct((N, D), jnp.float32),
    )(rel_adj, rel_emb, fused, proj_w, proj_b2)


def _forward(ent, rel_emb, rel_adj, edge, gat_w, gat_asrc, gat_adst,
             wq, wk, wv, proj_w, proj_b2):
    srcp = edge[0].astype(jnp.int32)
    dstp = edge[1].astype(jnp.int32)
    bsrc, bdst, cnts = _sc_partition(srcp, dstp)
    xs = [ent]
    x = ent
    for l in range(NLAYERS):
        hcat, aldt = _hidden_al(x, gat_w[l], gat_asrc[l], gat_adst[l])
        pay, wsum = _sc_edge_agg(hcat, aldt, bsrc, bdst, cnts)
        x = _combine(pay, wsum)[:N]
        xs.append(x)
    fused = _mha(xs[0], xs[1], xs[2], wq, wk, wv)
    return _relproj(rel_adj, rel_emb, fused, proj_w, proj_b2)


def kernel(ent_sr, ent_tg, rel_emb_sr, rel_emb_tg, rel_adj_sr, rel_adj_tg,
           gat_W, gat_asrc, gat_adst, Wq, Wk, Wv, proj_W, proj_b,
           edge_sr, edge_tg):
    pb = proj_b.reshape(1, D)
    sr = _forward(ent_sr, rel_emb_sr, rel_adj_sr, edge_sr,
                  gat_W, gat_asrc, gat_adst, Wq, Wk, Wv, proj_W, pb)
    tg = _forward(ent_tg, rel_emb_tg, rel_adj_tg, edge_tg,
                  gat_W, gat_asrc, gat_adst, Wq, Wk, Wv, proj_W, pb)
    return (sr, tg)


# speedup vs baseline: 1.6413x; 1.3001x over previous
import functools

import jax
import jax.numpy as jnp
from jax import lax
from jax.experimental import pallas as pl
from jax.experimental.pallas import tpu as pltpu
from jax.experimental.pallas import tpu_sc as plsc

N = 10000
D = 128
H = 2
E = 160000
RN = 1000
R = 3
NLAYERS = 2

LANES = 16
NTILES = 32
ROWS = 10112
NPB = ROWS // NTILES
PROC = 96
CAPP = 5120
EPPT = 4992
DIVM = 26547
DIVS = 23


HW = 272


def _hal_body(x_ref, w_ref, asrc_ref, adst_ref, h_ref, al_ref):
    x = x_ref[...]
    for h in range(H):
        hh = jnp.dot(x, w_ref[h], preferred_element_type=jnp.float32)
        h_ref[:, h * D:(h + 1) * D] = hh
        h_ref[:, H * D + h:H * D + h + 1] = lax.dot_general(
            hh, asrc_ref[h:h + 1, :], (((1,), (1,)), ((), ())),
            preferred_element_type=jnp.float32)
        al_ref[:, h:h + 1] = lax.dot_general(
            hh, adst_ref[h:h + 1, :], (((1,), (1,)), ((), ())),
            preferred_element_type=jnp.float32)


def _hidden_al(x, gw, gas, gad):
    bn = 1000
    return pl.pallas_call(
        _hal_body,
        grid=(N // bn,),
        in_specs=[pl.BlockSpec((bn, D), lambda i: (i, 0)),
                  pl.BlockSpec((H, D, D), lambda i: (0, 0, 0)),
                  pl.BlockSpec((H, D), lambda i: (0, 0)),
                  pl.BlockSpec((H, D), lambda i: (0, 0))],
        out_specs=[pl.BlockSpec((bn, HW), lambda i: (i, 0)),
                   pl.BlockSpec((bn, LANES), lambda i: (i, 0))],
        out_shape=[jax.ShapeDtypeStruct((N, HW), jnp.float32),
                   jax.ShapeDtypeStruct((N, LANES), jnp.float32)],
    )(x, gw, gas, gad)


def _sc_partition(src, dst):
    mesh = plsc.VectorSubcoreMesh(core_axis_name="c", subcore_axis_name="s")

    @functools.partial(
        pl.kernel,
        mesh=mesh,
        out_type=(jax.ShapeDtypeStruct((NTILES * NTILES * CAPP,), jnp.int32),
                  jax.ShapeDtypeStruct((NTILES * NTILES * CAPP,), jnp.int32),
                  jax.ShapeDtypeStruct((NTILES * NTILES,), jnp.int32)),
        compiler_params=pltpu.CompilerParams(needs_layout_passes=False,
                                             use_tc_tiling_on_sc=False),
        scratch_types=[
            pltpu.VMEM((EPPT,), jnp.int32),
            pltpu.VMEM((EPPT,), jnp.int32),
            pltpu.VMEM((LANES,), jnp.int32),
            pltpu.VMEM((LANES,), jnp.int32),
            pltpu.VMEM((NTILES * 64,), jnp.int32),
            pltpu.VMEM((NTILES * 64,), jnp.int32),
            pltpu.VMEM((NTILES,), jnp.int32),
            pltpu.VMEM((NTILES,), jnp.int32),
            pltpu.SemaphoreType.DMA,
        ],
    )
    def k(src_hbm, dst_hbm, bsrc_hbm, bdst_hbm, cnts_hbm,
          srcb, dstb, xsrc, xdst, stgs, stgd, scnt, wrt, semi):
        cid = lax.axis_index("c")
        sid = lax.axis_index("s")
        wid = cid * 16 + sid
        iota = lax.iota(jnp.int32, LANES)
        zi = jnp.zeros((LANES,), jnp.int32)

        for gz in range(NTILES * 64 // LANES):
            stgs[pl.ds(gz * LANES, LANES)] = zi
            stgd[pl.ds(gz * LANES, LANES)] = zi
        scnt[pl.ds(0, LANES)] = zi
        scnt[pl.ds(LANES, LANES)] = zi
        wrt[pl.ds(0, LANES)] = zi
        wrt[pl.ds(LANES, LANES)] = zi

        def do_group(sidx, didx):
            bv = lax.shift_right_logical(didx * DIVM, DIVS)
            for b in range(NTILES):
                grp = (b // LANES) * LANES
                ln = b % LANES
                m = bv == b
                c = plsc.all_reduce_population_count(m)[0]
                scv = scnt[pl.ds(grp, LANES)]
                sc_b = scv[ln]
                plsc.store_compressed(stgs.at[pl.ds(b * 64 + sc_b, LANES)],
                                      sidx, mask=m)
                plsc.store_compressed(stgd.at[pl.ds(b * 64 + sc_b, LANES)],
                                      didx, mask=m)
                nc = sc_b + c

                @pl.when(nc >= 48)
                def _():
                    wv = wrt[pl.ds(grp, LANES)]
                    w_b = wv[ln]
                    base = pl.multiple_of((b * NTILES + wid) * CAPP + w_b, 8)
                    pltpu.sync_copy(stgs.at[pl.ds(b * 64, 48)],
                                    bsrc_hbm.at[pl.ds(base, 48)])
                    pltpu.sync_copy(stgd.at[pl.ds(b * 64, 48)],
                                    bdst_hbm.at[pl.ds(base, 48)])
                    stgs[pl.ds(b * 64, LANES)] = stgs[pl.ds(b * 64 + 48,
                                                            LANES)]
                    stgd[pl.ds(b * 64, LANES)] = stgd[pl.ds(b * 64 + 48,
                                                            LANES)]
                    wrt[pl.ds(grp, LANES)] = jnp.where(iota == ln,
                                                       wv + 48, wv)
                nc2 = jnp.where(nc >= 48, nc - 48, nc)
                scv2 = scnt[pl.ds(grp, LANES)]
                scnt[pl.ds(grp, LANES)] = jnp.where(iota == ln, nc2, scv2)

        ls = pltpu.async_copy(src_hbm.at[pl.ds(pl.multiple_of(wid * EPPT, 8), EPPT)], srcb, semi)
        ld = pltpu.async_copy(dst_hbm.at[pl.ds(pl.multiple_of(wid * EPPT, 8), EPPT)], dstb, semi)
        ls.wait()
        ld.wait()

        def group_loop(g, _):
            sl = pl.ds(g * LANES, LANES)
            do_group(srcb[sl], dstb[sl])
            return 0
        lax.fori_loop(0, EPPT // LANES, group_loop, 0)

        @pl.when(wid < (E - NTILES * EPPT) // LANES)
        def _():
            xs = pltpu.async_copy(
                src_hbm.at[pl.ds(pl.multiple_of(NTILES * EPPT + wid * LANES, 8), LANES)],
                xsrc, semi)
            xd = pltpu.async_copy(
                dst_hbm.at[pl.ds(pl.multiple_of(NTILES * EPPT + wid * LANES, 8), LANES)],
                xdst, semi)
            xs.wait()
            xd.wait()
            do_group(xsrc[pl.ds(0, LANES)], xdst[pl.ds(0, LANES)])

        for b in range(NTILES):
            grp = (b // LANES) * LANES
            ln = b % LANES
            scv = scnt[pl.ds(grp, LANES)]
            sc_b = scv[ln]

            @pl.when(sc_b > 0)
            def _():
                wv = wrt[pl.ds(grp, LANES)]
                w_b = wv[ln]
                base = pl.multiple_of((b * NTILES + wid) * CAPP + w_b, 8)
                pltpu.sync_copy(stgs.at[pl.ds(b * 64, 48)],
                                bsrc_hbm.at[pl.ds(base, 48)])
                pltpu.sync_copy(stgd.at[pl.ds(b * 64, 48)],
                                bdst_hbm.at[pl.ds(base, 48)])
        c0 = wrt[pl.ds(0, LANES)] + scnt[pl.ds(0, LANES)]
        c1 = wrt[pl.ds(LANES, LANES)] + scnt[pl.ds(LANES, LANES)]
        scnt[pl.ds(0, LANES)] = c0
        scnt[pl.ds(LANES, LANES)] = c1
        pltpu.sync_copy(scnt, cnts_hbm.at[pl.ds(pl.multiple_of(wid * NTILES, 8), NTILES)])

    return k(src, dst)


def _sc_edge_agg(hcat, aldt, bsrc, bdst, cnts):
    mesh = plsc.VectorSubcoreMesh(core_axis_name="c", subcore_axis_name="s")

    @functools.partial(
        pl.kernel,
        mesh=mesh,
        out_type=(jax.ShapeDtypeStruct((H, ROWS, D), jnp.float32),
                  jax.ShapeDtypeStruct((ROWS, LANES), jnp.float32)),
        compiler_params=pltpu.CompilerParams(needs_layout_passes=False,
                                             use_tc_tiling_on_sc=False),
        scratch_types=[
            pltpu.VMEM((NPB, D), jnp.float32),
            pltpu.VMEM((NPB, D), jnp.float32),
            pltpu.VMEM((NPB, LANES), jnp.float32),
            pltpu.VMEM((NPB, LANES), jnp.float32),
            pltpu.VMEM((NTILES * NTILES,), jnp.int32),
            pltpu.VMEM((2 * PROC,), jnp.int32),
            pltpu.VMEM((2 * PROC,), jnp.int32),
            pltpu.VMEM((PROC,), jnp.int32),
            pltpu.VMEM((PROC,), jnp.int32),
            pltpu.VMEM((PROC,), jnp.int32),
            pltpu.VMEM((PROC,), jnp.float32),
            pltpu.VMEM((PROC,), jnp.float32),
            pltpu.VMEM((PROC, HW), jnp.float32),
            pltpu.SemaphoreType.DMA,
            pltpu.SemaphoreType.DMA,
        ],
    )
    def k(h_hbm, aldt_hbm, bsrc_hbm, bdst_hbm, cnts_hbm,
          pay_hbm, wsum_hbm,
          acc0, acc1, accw, albuf, cntv, ps, pd, ssrc, sdst, dlb,
          w0b, w1b, rowsb, semr, semi):
        cid = lax.axis_index("c")
        sid = lax.axis_index("s")
        wid = cid * 16 + sid
        lo = wid * NPB
        iota = lax.iota(jnp.int32, LANES)
        zf = jnp.zeros((LANES,), jnp.float32)
        e0 = (iota == 0).astype(jnp.float32)
        e1 = (iota == 1).astype(jnp.float32)
        c256 = jnp.zeros((LANES,), jnp.int32) + (H * D)
        c257 = jnp.zeros((LANES,), jnp.int32) + (H * D + 1)
        cz = jnp.zeros((LANES,), jnp.int32)
        co = jnp.zeros((LANES,), jnp.int32) + 1

        cg = pltpu.async_copy(cnts_hbm, cntv, semi)
        ag = pltpu.async_copy(aldt_hbm.at[pl.ds(lo, NPB)], albuf, semr)

        def zacc(i, _):
            for dpart in range(D // LANES):
                acc0[i, pl.ds(dpart * LANES, LANES)] = zf
                acc1[i, pl.ds(dpart * LANES, LANES)] = zf
            accw[i, :] = zf
            return 0
        lax.fori_loop(0, NPB, zacc, 0)
        cg.wait()
        ag.wait()

        def process_batch(count):
            gr = pltpu.async_copy(h_hbm.at[ssrc], rowsb, semr)
            for g in range(PROC // LANES):
                sl = pl.ds(g * LANES, LANES)
                dl = sdst[sl] - lo
                dlb[sl] = jnp.minimum(jnp.maximum(dl, 0), NPB - 1)
            gr.wait()
            for g in range(PROC // LANES):
                sl = pl.ds(g * LANES, LANES)
                ei = g * LANES + iota
                valid = ei < count
                dl16 = dlb[sl]
                a0 = plsc.load_gather(rowsb, [ei, c256])
                a1 = plsc.load_gather(rowsb, [ei, c257])
                b0 = plsc.load_gather(albuf, [dl16, cz])
                b1 = plsc.load_gather(albuf, [dl16, co])
                x0 = a0 + b0
                w0b[sl] = jnp.where(valid,
                                    jnp.exp(jnp.maximum(x0, 0.2 * x0)), 0.0)
                x1 = a1 + b1
                w1b[sl] = jnp.where(valid,
                                    jnp.exp(jnp.maximum(x1, 0.2 * x1)), 0.0)

            def accum(g, _):
                sl = pl.ds(g * LANES, LANES)
                dl16 = dlb[sl]
                w016 = w0b[sl]
                w116 = w1b[sl]
                for j in range(LANES):
                    i = g * LANES + j
                    dlj = dl16[j]
                    w0j = w016[j]
                    w1j = w116[j]
                    for dpart in range(D // LANES):
                        c = pl.ds(dpart * LANES, LANES)
                        plsc.addupdate(acc0.at[dlj, c], w0j * rowsb[i, c])
                        plsc.addupdate(acc1.at[dlj, c],
                                       w1j * rowsb[i, pl.ds(D + dpart * LANES,
                                                            LANES)])
                    plsc.addupdate(accw.at[dlj, :], w0j * e0 + w1j * e1)
                return 0
            lax.fori_loop(0, PROC // LANES, accum, 0)

        def per_p(p, _):
            ci = plsc.load_gather(
                cntv, [jnp.zeros((LANES,), jnp.int32) + (p * NTILES + wid)])
            cnt = ci[0]
            base = (wid * NTILES + p) * CAPP
            nb = (cnt + PROC - 1) // PROC
            lp = pltpu.async_copy(
                bsrc_hbm.at[pl.ds(pl.multiple_of(base, 8), 2 * PROC)],
                ps, semi)
            ld = pltpu.async_copy(
                bdst_hbm.at[pl.ds(pl.multiple_of(base, 8), 2 * PROC)],
                pd, semi)
            lp.wait()
            ld.wait()

            def per_batch(kb, _):
                @pl.when(kb < 2)
                def _():
                    for g in range(PROC // LANES):
                        sl = pl.ds(g * LANES, LANES)
                        psl = pl.ds(kb * PROC + g * LANES, LANES)
                        ssrc[sl] = jnp.minimum(jnp.maximum(ps[psl], 0), N - 1)
                        sdst[sl] = jnp.minimum(jnp.maximum(pd[psl], 0), N - 1)

                @pl.when(kb >= 2)
                def _():
                    off = pl.multiple_of(base + kb * PROC, 8)
                    ls2 = pltpu.async_copy(bsrc_hbm.at[pl.ds(off, PROC)],
                                           ssrc, semi)
                    ld2 = pltpu.async_copy(bdst_hbm.at[pl.ds(off, PROC)],
                                           sdst, semi)
                    ls2.wait()
                    ld2.wait()
                    for g in range(PROC // LANES):
                        sl = pl.ds(g * LANES, LANES)
                        ssrc[sl] = jnp.minimum(jnp.maximum(ssrc[sl], 0), N - 1)
                        sdst[sl] = jnp.minimum(jnp.maximum(sdst[sl], 0), N - 1)
                process_batch(cnt - kb * PROC)
                return 0
            lax.fori_loop(0, nb, per_batch, 0)
            return 0
        lax.fori_loop(0, NTILES, per_p, 0)

        pltpu.sync_copy(acc0, pay_hbm.at[0, pl.ds(lo, NPB)])
        pltpu.sync_copy(acc1, pay_hbm.at[1, pl.ds(lo, NPB)])
        pltpu.sync_copy(accw, wsum_hbm.at[pl.ds(lo, NPB)])

    return k(hcat, aldt, bsrc, bdst, cnts)


def _comb_body(p_ref, w_ref, o_ref):
    accm = None
    for h in range(H):
        num = p_ref[h]
        den = w_ref[:, h:h + 1]
        v = num / (den + 1e-16)
        e = jnp.where(v > 0, v, jnp.exp(jnp.minimum(v, 0.0)) - 1.0)
        accm = e if accm is None else accm + e
    m = accm * (1.0 / H)
    nrm = jnp.sqrt(jnp.sum(m * m, axis=1, keepdims=True))
    o_ref[...] = m / (nrm + 1e-12)


def _combine(pay, wsum):
    bn = 632
    return pl.pallas_call(
        _comb_body,
        grid=(ROWS // bn,),
        in_specs=[pl.BlockSpec((H, bn, D), lambda i: (0, i, 0)),
                  pl.BlockSpec((bn, LANES), lambda i: (i, 0))],
        out_specs=pl.BlockSpec((bn, D), lambda i: (i, 0)),
        out_shape=jax.ShapeDtypeStruct((ROWS, D), jnp.float32),
    )(pay, wsum)


_INV_SQRT_D = 0.08838834764831845


def _mha_body(x0_ref, x1_ref, x2_ref, wq_ref, wk_ref, wv_ref, o_ref):
    xs = [x0_ref[...], x1_ref[...], x2_ref[...]]
    for h in range(H):
        q = [jnp.dot(x, wq_ref[h], preferred_element_type=jnp.float32)
             for x in xs]
        kk = [jnp.dot(x, wk_ref[h], preferred_element_type=jnp.float32)
              for x in xs]
        vv = [jnp.dot(x, wv_ref[h], preferred_element_type=jnp.float32)
              for x in xs]
        osum = None
        for r in range(R):
            att = [jnp.sum(q[r] * kk[s], axis=1, keepdims=True) * _INV_SQRT_D
                   for s in range(R)]
            m = jnp.maximum(jnp.maximum(att[0], att[1]), att[2])
            ee = [jnp.exp(a - m) for a in att]
            den = ee[0] + ee[1] + ee[2]
            o_r = (ee[0] * vv[0] + ee[1] * vv[1] + ee[2] * vv[2]) / den
            osum = o_r if osum is None else osum + o_r
        o_ref[:, h * D:(h + 1) * D] = osum * (1.0 / R)


def _mha(x0, x1, x2, wq, wk, wv):
    bn = 1000
    return pl.pallas_call(
        _mha_body,
        grid=(N // bn,),
        in_specs=[pl.BlockSpec((bn, D), lambda i: (i, 0)),
                  pl.BlockSpec((bn, D), lambda i: (i, 0)),
                  pl.BlockSpec((bn, D), lambda i: (i, 0)),
                  pl.BlockSpec((H, D, D), lambda i: (0, 0, 0)),
                  pl.BlockSpec((H, D, D), lambda i: (0, 0, 0)),
                  pl.BlockSpec((H, D, D), lambda i: (0, 0, 0))],
        out_specs=pl.BlockSpec((bn, H * D), lambda i: (i, 0)),
        out_shape=jax.ShapeDtypeStruct((N, H * D), jnp.float32),
    )(x0, x1, x2, wq, wk, wv)


def _proj_body(adj_ref, emb_ref, fused_ref, w_ref, b_ref, o_ref):
    adj = adj_ref[...]
    rs = jnp.sum(adj, axis=1, keepdims=True)
    ragg = jnp.dot(adj, emb_ref[...],
                   preferred_element_type=jnp.float32) / (rs + 1e-5)
    f = jnp.dot(fused_ref[...], w_ref[:H * D, :],
                preferred_element_type=jnp.float32)
    g = jnp.dot(ragg, w_ref[H * D:, :], preferred_element_type=jnp.float32)
    o_ref[...] = jnp.maximum(f + g + b_ref[...], 0.0)


def _relproj(rel_adj, rel_emb, fused, proj_w, proj_b2):
    bn = 1000
    return pl.pallas_call(
        _proj_body,
        grid=(N // bn,),
        in_specs=[pl.BlockSpec((bn, RN), lambda i: (i, 0)),
                  pl.BlockSpec((RN, D), lambda i: (0, 0)),
                  pl.BlockSpec((bn, H * D), lambda i: (i, 0)),
                  pl.BlockSpec((H * D + D, D), lambda i: (0, 0)),
                  pl.BlockSpec((1, D), lambda i: (0, 0))],
        out_specs=pl.BlockSpec((bn, D), lambda i: (i, 0)),
        out_shape=jax.ShapeDtypeStruct((N, D), jnp.float32),
    )(rel_adj, rel_emb, fused, proj_w, proj_b2)


def _forward(ent, rel_emb, rel_adj, edge, gat_w, gat_asrc, gat_adst,
             wq, wk, wv, proj_w, proj_b2):
    srcp = edge[0].astype(jnp.int32)
    dstp = edge[1].astype(jnp.int32)
    bsrc, bdst, cnts = _sc_partition(srcp, dstp)
    xs = [ent]
    x = ent
    for l in range(NLAYERS):
        hcat, aldt = _hidden_al(x, gat_w[l], gat_asrc[l], gat_adst[l])
        pay, wsum = _sc_edge_agg(hcat, aldt, bsrc, bdst, cnts)
        x = _combine(pay, wsum)[:N]
        xs.append(x)
    fused = _mha(xs[0], xs[1], xs[2], wq, wk, wv)
    return _relproj(rel_adj, rel_emb, fused, proj_w, proj_b2)


def kernel(ent_sr, ent_tg, rel_emb_sr, rel_emb_tg, rel_adj_sr, rel_adj_tg,
           gat_W, gat_asrc, gat_adst, Wq, Wk, Wv, proj_W, proj_b,
           edge_sr, edge_tg):
    pb = proj_b.reshape(1, D)
    sr = _forward(ent_sr, rel_emb_sr, rel_adj_sr, edge_sr,
                  gat_W, gat_asrc, gat_adst, Wq, Wk, Wv, proj_W, pb)
    tg = _forward(ent_tg, rel_emb_tg, rel_adj_tg, edge_tg,
                  gat_W, gat_asrc, gat_adst, Wq, Wk, Wv, proj_W, pb)
    return (sr, tg)


# speedup vs baseline: 1.6875x; 1.0281x over previous
import functools

import jax
import jax.numpy as jnp
from jax import lax
from jax.experimental import pallas as pl
from jax.experimental.pallas import tpu as pltpu
from jax.experimental.pallas import tpu_sc as plsc

N = 10000
D = 128
H = 2
E = 160000
RN = 1000
R = 3
NLAYERS = 2

LANES = 16
NTILES = 32
ROWS = 10112
NPB = ROWS // NTILES
PROC = 96
CAPP = 5120
EPPT = 4992
DIVM = 26547
DIVS = 23


HW = 272


def _hal_body(x_ref, w_ref, asrc_ref, adst_ref, h_ref, al_ref):
    x = x_ref[...]
    for h in range(H):
        hh = jnp.dot(x, w_ref[h], preferred_element_type=jnp.float32)
        h_ref[:, h * D:(h + 1) * D] = hh
        h_ref[:, H * D + h:H * D + h + 1] = lax.dot_general(
            hh, asrc_ref[h:h + 1, :], (((1,), (1,)), ((), ())),
            preferred_element_type=jnp.float32)
        al_ref[:, h:h + 1] = lax.dot_general(
            hh, adst_ref[h:h + 1, :], (((1,), (1,)), ((), ())),
            preferred_element_type=jnp.float32)


def _hidden_al(x, gw, gas, gad):
    bn = 1000
    return pl.pallas_call(
        _hal_body,
        grid=(N // bn,),
        in_specs=[pl.BlockSpec((bn, D), lambda i: (i, 0)),
                  pl.BlockSpec((H, D, D), lambda i: (0, 0, 0)),
                  pl.BlockSpec((H, D), lambda i: (0, 0)),
                  pl.BlockSpec((H, D), lambda i: (0, 0))],
        out_specs=[pl.BlockSpec((bn, HW), lambda i: (i, 0)),
                   pl.BlockSpec((bn, LANES), lambda i: (i, 0))],
        out_shape=[jax.ShapeDtypeStruct((N, HW), jnp.float32),
                   jax.ShapeDtypeStruct((N, LANES), jnp.float32)],
    )(x, gw, gas, gad)


def _sc_partition(src, dst):
    mesh = plsc.VectorSubcoreMesh(core_axis_name="c", subcore_axis_name="s")

    @functools.partial(
        pl.kernel,
        mesh=mesh,
        out_type=(jax.ShapeDtypeStruct((NTILES * NTILES * CAPP,), jnp.int32),
                  jax.ShapeDtypeStruct((NTILES * NTILES * CAPP,), jnp.int32),
                  jax.ShapeDtypeStruct((NTILES * NTILES,), jnp.int32)),
        compiler_params=pltpu.CompilerParams(needs_layout_passes=False,
                                             use_tc_tiling_on_sc=False),
        scratch_types=[
            pltpu.VMEM((EPPT,), jnp.int32),
            pltpu.VMEM((EPPT,), jnp.int32),
            pltpu.VMEM((LANES,), jnp.int32),
            pltpu.VMEM((LANES,), jnp.int32),
            pltpu.VMEM((NTILES * 64,), jnp.int32),
            pltpu.VMEM((NTILES * 64,), jnp.int32),
            pltpu.VMEM((NTILES,), jnp.int32),
            pltpu.VMEM((NTILES,), jnp.int32),
            pltpu.SemaphoreType.DMA,
        ],
    )
    def k(src_hbm, dst_hbm, bsrc_hbm, bdst_hbm, cnts_hbm,
          srcb, dstb, xsrc, xdst, stgs, stgd, scnt, wrt, semi):
        cid = lax.axis_index("c")
        sid = lax.axis_index("s")
        wid = cid * 16 + sid
        iota = lax.iota(jnp.int32, LANES)
        zi = jnp.zeros((LANES,), jnp.int32)

        for gz in range(NTILES * 64 // LANES):
            stgs[pl.ds(gz * LANES, LANES)] = zi
            stgd[pl.ds(gz * LANES, LANES)] = zi
        scnt[pl.ds(0, LANES)] = zi
        scnt[pl.ds(LANES, LANES)] = zi
        wrt[pl.ds(0, LANES)] = zi
        wrt[pl.ds(LANES, LANES)] = zi

        def do_group(sidx, didx):
            bv = lax.shift_right_logical(didx * DIVM, DIVS)
            for b in range(NTILES):
                grp = (b // LANES) * LANES
                ln = b % LANES
                m = bv == b
                c = plsc.all_reduce_population_count(m)[0]
                scv = scnt[pl.ds(grp, LANES)]
                sc_b = scv[ln]
                plsc.store_compressed(stgs.at[pl.ds(b * 64 + sc_b, LANES)],
                                      sidx, mask=m)
                plsc.store_compressed(stgd.at[pl.ds(b * 64 + sc_b, LANES)],
                                      didx, mask=m)
                nc = sc_b + c

                @pl.when(nc >= 48)
                def _():
                    wv = wrt[pl.ds(grp, LANES)]
                    w_b = wv[ln]
                    base = pl.multiple_of((b * NTILES + wid) * CAPP + w_b, 8)
                    pltpu.sync_copy(stgs.at[pl.ds(b * 64, 48)],
                                    bsrc_hbm.at[pl.ds(base, 48)])
                    pltpu.sync_copy(stgd.at[pl.ds(b * 64, 48)],
                                    bdst_hbm.at[pl.ds(base, 48)])
                    stgs[pl.ds(b * 64, LANES)] = stgs[pl.ds(b * 64 + 48,
                                                            LANES)]
                    stgd[pl.ds(b * 64, LANES)] = stgd[pl.ds(b * 64 + 48,
                                                            LANES)]
                    wrt[pl.ds(grp, LANES)] = jnp.where(iota == ln,
                                                       wv + 48, wv)
                nc2 = jnp.where(nc >= 48, nc - 48, nc)
                scv2 = scnt[pl.ds(grp, LANES)]
                scnt[pl.ds(grp, LANES)] = jnp.where(iota == ln, nc2, scv2)

        ls = pltpu.async_copy(src_hbm.at[pl.ds(pl.multiple_of(wid * EPPT, 8), EPPT)], srcb, semi)
        ld = pltpu.async_copy(dst_hbm.at[pl.ds(pl.multiple_of(wid * EPPT, 8), EPPT)], dstb, semi)
        ls.wait()
        ld.wait()

        def group_loop(g, _):
            sl = pl.ds(g * LANES, LANES)
            do_group(srcb[sl], dstb[sl])
            return 0
        lax.fori_loop(0, EPPT // LANES, group_loop, 0)

        @pl.when(wid < (E - NTILES * EPPT) // LANES)
        def _():
            xs = pltpu.async_copy(
                src_hbm.at[pl.ds(pl.multiple_of(NTILES * EPPT + wid * LANES, 8), LANES)],
                xsrc, semi)
            xd = pltpu.async_copy(
                dst_hbm.at[pl.ds(pl.multiple_of(NTILES * EPPT + wid * LANES, 8), LANES)],
                xdst, semi)
            xs.wait()
            xd.wait()
            do_group(xsrc[pl.ds(0, LANES)], xdst[pl.ds(0, LANES)])

        for b in range(NTILES):
            grp = (b // LANES) * LANES
            ln = b % LANES
            scv = scnt[pl.ds(grp, LANES)]
            sc_b = scv[ln]

            @pl.when(sc_b > 0)
            def _():
                wv = wrt[pl.ds(grp, LANES)]
                w_b = wv[ln]
                base = pl.multiple_of((b * NTILES + wid) * CAPP + w_b, 8)
                pltpu.sync_copy(stgs.at[pl.ds(b * 64, 48)],
                                bsrc_hbm.at[pl.ds(base, 48)])
                pltpu.sync_copy(stgd.at[pl.ds(b * 64, 48)],
                                bdst_hbm.at[pl.ds(base, 48)])
        c0 = wrt[pl.ds(0, LANES)] + scnt[pl.ds(0, LANES)]
        c1 = wrt[pl.ds(LANES, LANES)] + scnt[pl.ds(LANES, LANES)]
        scnt[pl.ds(0, LANES)] = c0
        scnt[pl.ds(LANES, LANES)] = c1
        pltpu.sync_copy(scnt, cnts_hbm.at[pl.ds(pl.multiple_of(wid * NTILES, 8), NTILES)])

    return k(src, dst)


def _sc_edge_agg(hcat, aldt, bsrc, bdst, cnts):
    mesh = plsc.VectorSubcoreMesh(core_axis_name="c", subcore_axis_name="s")

    @functools.partial(
        pl.kernel,
        mesh=mesh,
        out_type=(jax.ShapeDtypeStruct((H, ROWS, D), jnp.float32),
                  jax.ShapeDtypeStruct((ROWS, LANES), jnp.float32)),
        compiler_params=pltpu.CompilerParams(needs_layout_passes=False,
                                             use_tc_tiling_on_sc=False),
        scratch_types=[
            pltpu.VMEM((NPB, D), jnp.float32),
            pltpu.VMEM((NPB, D), jnp.float32),
            pltpu.VMEM((NPB, LANES), jnp.float32),
            pltpu.VMEM((NPB, LANES), jnp.float32),
            pltpu.VMEM((NTILES * NTILES,), jnp.int32),
            pltpu.VMEM((2 * PROC,), jnp.int32),
            pltpu.VMEM((2 * PROC,), jnp.int32),
            pltpu.VMEM((PROC,), jnp.int32),
            pltpu.VMEM((PROC,), jnp.int32),
            pltpu.VMEM((PROC,), jnp.int32),
            pltpu.VMEM((PROC,), jnp.float32),
            pltpu.VMEM((PROC,), jnp.float32),
            pltpu.VMEM((PROC, HW), jnp.float32),
            pltpu.SemaphoreType.DMA,
            pltpu.SemaphoreType.DMA,
        ],
    )
    def k(h_hbm, aldt_hbm, bsrc_hbm, bdst_hbm, cnts_hbm,
          pay_hbm, wsum_hbm,
          acc0, acc1, accw, albuf, cntv, ps, pd, ssrc, sdst, dlb,
          w0b, w1b, rowsb, semr, semi):
        cid = lax.axis_index("c")
        sid = lax.axis_index("s")
        wid = cid * 16 + sid
        lo = wid * NPB
        iota = lax.iota(jnp.int32, LANES)
        zf = jnp.zeros((LANES,), jnp.float32)
        e0 = (iota == 0).astype(jnp.float32)
        e1 = (iota == 1).astype(jnp.float32)
        c256 = jnp.zeros((LANES,), jnp.int32) + (H * D)
        c257 = jnp.zeros((LANES,), jnp.int32) + (H * D + 1)
        cz = jnp.zeros((LANES,), jnp.int32)
        co = jnp.zeros((LANES,), jnp.int32) + 1

        cg = pltpu.async_copy(cnts_hbm, cntv, semi)
        ag = pltpu.async_copy(aldt_hbm.at[pl.ds(lo, NPB)], albuf, semr)

        def zacc(i, _):
            for dpart in range(D // LANES):
                acc0[i, pl.ds(dpart * LANES, LANES)] = zf
                acc1[i, pl.ds(dpart * LANES, LANES)] = zf
            accw[i, :] = zf
            return 0
        lax.fori_loop(0, NPB, zacc, 0)
        cg.wait()
        ag.wait()

        HB = PROC // 2

        def process_batch(count):
            ga = pltpu.async_copy(h_hbm.at[ssrc.at[pl.ds(0, HB)]],
                                  rowsb.at[pl.ds(0, HB)], semr)
            for g in range(PROC // LANES):
                sl = pl.ds(g * LANES, LANES)
                dl = sdst[sl] - lo
                dlb[sl] = jnp.minimum(jnp.maximum(dl, 0), NPB - 1)
            ga.wait()
            gb = pltpu.async_copy(h_hbm.at[ssrc.at[pl.ds(HB, HB)]],
                                  rowsb.at[pl.ds(HB, HB)], semr)

            def half(h0g):
                for g in range(h0g, h0g + PROC // LANES // 2):
                    sl = pl.ds(g * LANES, LANES)
                    ei = g * LANES + iota
                    valid = ei < count
                    dl16 = dlb[sl]
                    a0 = plsc.load_gather(rowsb, [ei, c256])
                    a1 = plsc.load_gather(rowsb, [ei, c257])
                    b0 = plsc.load_gather(albuf, [dl16, cz])
                    b1 = plsc.load_gather(albuf, [dl16, co])
                    x0 = a0 + b0
                    w0b[sl] = jnp.where(
                        valid, jnp.exp(jnp.maximum(x0, 0.2 * x0)), 0.0)
                    x1 = a1 + b1
                    w1b[sl] = jnp.where(
                        valid, jnp.exp(jnp.maximum(x1, 0.2 * x1)), 0.0)

                def accum(g, _):
                    sl = pl.ds(g * LANES, LANES)
                    dl16 = dlb[sl]
                    w016 = w0b[sl]
                    w116 = w1b[sl]
                    for j in range(LANES):
                        i = g * LANES + j
                        dlj = dl16[j]
                        w0j = w016[j]
                        w1j = w116[j]
                        for dpart in range(D // LANES):
                            c = pl.ds(dpart * LANES, LANES)
                            plsc.addupdate(acc0.at[dlj, c], w0j * rowsb[i, c])
                            plsc.addupdate(
                                acc1.at[dlj, c],
                                w1j * rowsb[i, pl.ds(D + dpart * LANES,
                                                     LANES)])
                        plsc.addupdate(accw.at[dlj, :], w0j * e0 + w1j * e1)
                    return 0
                lax.fori_loop(h0g, h0g + PROC // LANES // 2, accum, 0)
            half(0)
            gb.wait()
            half(PROC // LANES // 2)

        def per_p(p, _):
            ci = plsc.load_gather(
                cntv, [jnp.zeros((LANES,), jnp.int32) + (p * NTILES + wid)])
            cnt = ci[0]
            base = (wid * NTILES + p) * CAPP
            nb = (cnt + PROC - 1) // PROC
            lp = pltpu.async_copy(
                bsrc_hbm.at[pl.ds(pl.multiple_of(base, 8), 2 * PROC)],
                ps, semi)
            ld = pltpu.async_copy(
                bdst_hbm.at[pl.ds(pl.multiple_of(base, 8), 2 * PROC)],
                pd, semi)
            lp.wait()
            ld.wait()

            def per_batch(kb, _):
                @pl.when(kb < 2)
                def _():
                    for g in range(PROC // LANES):
                        sl = pl.ds(g * LANES, LANES)
                        psl = pl.ds(kb * PROC + g * LANES, LANES)
                        ssrc[sl] = jnp.minimum(jnp.maximum(ps[psl], 0), N - 1)
                        sdst[sl] = jnp.minimum(jnp.maximum(pd[psl], 0), N - 1)

                @pl.when(kb >= 2)
                def _():
                    off = pl.multiple_of(base + kb * PROC, 8)
                    ls2 = pltpu.async_copy(bsrc_hbm.at[pl.ds(off, PROC)],
                                           ssrc, semi)
                    ld2 = pltpu.async_copy(bdst_hbm.at[pl.ds(off, PROC)],
                                           sdst, semi)
                    ls2.wait()
                    ld2.wait()
                    for g in range(PROC // LANES):
                        sl = pl.ds(g * LANES, LANES)
                        ssrc[sl] = jnp.minimum(jnp.maximum(ssrc[sl], 0), N - 1)
                        sdst[sl] = jnp.minimum(jnp.maximum(sdst[sl], 0), N - 1)
                process_batch(cnt - kb * PROC)
                return 0
            lax.fori_loop(0, nb, per_batch, 0)
            return 0
        lax.fori_loop(0, NTILES, per_p, 0)

        pltpu.sync_copy(acc0, pay_hbm.at[0, pl.ds(lo, NPB)])
        pltpu.sync_copy(acc1, pay_hbm.at[1, pl.ds(lo, NPB)])
        pltpu.sync_copy(accw, wsum_hbm.at[pl.ds(lo, NPB)])

    return k(hcat, aldt, bsrc, bdst, cnts)


def _comb_body(p_ref, w_ref, o_ref):
    accm = None
    for h in range(H):
        num = p_ref[h]
        den = w_ref[:, h:h + 1]
        v = num / (den + 1e-16)
        e = jnp.where(v > 0, v, jnp.exp(jnp.minimum(v, 0.0)) - 1.0)
        accm = e if accm is None else accm + e
    m = accm * (1.0 / H)
    nrm = jnp.sqrt(jnp.sum(m * m, axis=1, keepdims=True))
    o_ref[...] = m / (nrm + 1e-12)


def _combine(pay, wsum):
    bn = 632
    return pl.pallas_call(
        _comb_body,
        grid=(ROWS // bn,),
        in_specs=[pl.BlockSpec((H, bn, D), lambda i: (0, i, 0)),
                  pl.BlockSpec((bn, LANES), lambda i: (i, 0))],
        out_specs=pl.BlockSpec((bn, D), lambda i: (i, 0)),
        out_shape=jax.ShapeDtypeStruct((ROWS, D), jnp.float32),
    )(pay, wsum)


_INV_SQRT_D = 0.08838834764831845


def _mha_body(x0_ref, x1_ref, x2_ref, wq_ref, wk_ref, wv_ref, o_ref):
    xs = [x0_ref[...], x1_ref[...], x2_ref[...]]
    for h in range(H):
        q = [jnp.dot(x, wq_ref[h], preferred_element_type=jnp.float32)
             for x in xs]
        kk = [jnp.dot(x, wk_ref[h], preferred_element_type=jnp.float32)
              for x in xs]
        vv = [jnp.dot(x, wv_ref[h], preferred_element_type=jnp.float32)
              for x in xs]
        osum = None
        for r in range(R):
            att = [jnp.sum(q[r] * kk[s], axis=1, keepdims=True) * _INV_SQRT_D
                   for s in range(R)]
            m = jnp.maximum(jnp.maximum(att[0], att[1]), att[2])
            ee = [jnp.exp(a - m) for a in att]
            den = ee[0] + ee[1] + ee[2]
            o_r = (ee[0] * vv[0] + ee[1] * vv[1] + ee[2] * vv[2]) / den
            osum = o_r if osum is None else osum + o_r
        o_ref[:, h * D:(h + 1) * D] = osum * (1.0 / R)


def _mha(x0, x1, x2, wq, wk, wv):
    bn = 1000
    return pl.pallas_call(
        _mha_body,
        grid=(N // bn,),
        in_specs=[pl.BlockSpec((bn, D), lambda i: (i, 0)),
                  pl.BlockSpec((bn, D), lambda i: (i, 0)),
                  pl.BlockSpec((bn, D), lambda i: (i, 0)),
                  pl.BlockSpec((H, D, D), lambda i: (0, 0, 0)),
                  pl.BlockSpec((H, D, D), lambda i: (0, 0, 0)),
                  pl.BlockSpec((H, D, D), lambda i: (0, 0, 0))],
        out_specs=pl.BlockSpec((bn, H * D), lambda i: (i, 0)),
        out_shape=jax.ShapeDtypeStruct((N, H * D), jnp.float32),
    )(x0, x1, x2, wq, wk, wv)


def _proj_body(adj_ref, emb_ref, fused_ref, w_ref, b_ref, o_ref):
    adj = adj_ref[...]
    rs = jnp.sum(adj, axis=1, keepdims=True)
    ragg = jnp.dot(adj, emb_ref[...],
                   preferred_element_type=jnp.float32) / (rs + 1e-5)
    f = jnp.dot(fused_ref[...], w_ref[:H * D, :],
                preferred_element_type=jnp.float32)
    g = jnp.dot(ragg, w_ref[H * D:, :], preferred_element_type=jnp.float32)
    o_ref[...] = jnp.maximum(f + g + b_ref[...], 0.0)


def _relproj(rel_adj, rel_emb, fused, proj_w, proj_b2):
    bn = 1000
    return pl.pallas_call(
        _proj_body,
        grid=(N // bn,),
        in_specs=[pl.BlockSpec((bn, RN), lambda i: (i, 0)),
                  pl.BlockSpec((RN, D), lambda i: (0, 0)),
                  pl.BlockSpec((bn, H * D), lambda i: (i, 0)),
                  pl.BlockSpec((H * D + D, D), lambda i: (0, 0)),
                  pl.BlockSpec((1, D), lambda i: (0, 0))],
        out_specs=pl.BlockSpec((bn, D), lambda i: (i, 0)),
        out_shape=jax.ShapeDtypeStruct((N, D), jnp.float32),
    )(rel_adj, rel_emb, fused, proj_w, proj_b2)


def _forward(ent, rel_emb, rel_adj, edge, gat_w, gat_asrc, gat_adst,
             wq, wk, wv, proj_w, proj_b2):
    srcp = edge[0].astype(jnp.int32)
    dstp = edge[1].astype(jnp.int32)
    bsrc, bdst, cnts = _sc_partition(srcp, dstp)
    xs = [ent]
    x = ent
    for l in range(NLAYERS):
        hcat, aldt = _hidden_al(x, gat_w[l], gat_asrc[l], gat_adst[l])
        pay, wsum = _sc_edge_agg(hcat, aldt, bsrc, bdst, cnts)
        x = _combine(pay, wsum)[:N]
        xs.append(x)
    fused = _mha(xs[0], xs[1], xs[2], wq, wk, wv)
    return _relproj(rel_adj, rel_emb, fused, proj_w, proj_b2)


def kernel(ent_sr, ent_tg, rel_emb_sr, rel_emb_tg, rel_adj_sr, rel_adj_tg,
           gat_W, gat_asrc, gat_adst, Wq, Wk, Wv, proj_W, proj_b,
           edge_sr, edge_tg):
    pb = proj_b.reshape(1, D)
    sr = _forward(ent_sr, rel_emb_sr, rel_adj_sr, edge_sr,
                  gat_W, gat_asrc, gat_adst, Wq, Wk, Wv, proj_W, pb)
    tg = _forward(ent_tg, rel_emb_tg, rel_adj_tg, edge_tg,
                  gat_W, gat_asrc, gat_adst, Wq, Wk, Wv, proj_W, pb)
    return (sr, tg)


# speedup vs baseline: 1.8777x; 1.1127x over previous
import functools

import jax
import jax.numpy as jnp
from jax import lax
from jax.experimental import pallas as pl
from jax.experimental.pallas import tpu as pltpu
from jax.experimental.pallas import tpu_sc as plsc

N = 10000
D = 128
H = 2
E = 160000
RN = 1000
R = 3
NLAYERS = 2

LANES = 16
NTILES = 32
CHUNK = 80
EPAD = 163840
EPT = EPAD // NTILES
NCHUNKS = EPT // CHUNK
ROWS = 10080
CH = 144
NTAB = 10016


def _hal_body(x_ref, w_ref, asrc_ref, adst_ref, h_ref, al_ref):
    x = x_ref[...]
    for h in range(H):
        hh = jnp.dot(x, w_ref[h], preferred_element_type=jnp.float32)
        h_ref[h] = hh
        al_ref[:, h:h + 1] = lax.dot_general(
            hh, asrc_ref[h:h + 1, :], (((1,), (1,)), ((), ())),
            preferred_element_type=jnp.float32)
        al_ref[:, H + h:H + h + 1] = lax.dot_general(
            hh, adst_ref[h:h + 1, :], (((1,), (1,)), ((), ())),
            preferred_element_type=jnp.float32)


def _hidden_al(x, gw, gas, gad):
    bn = 1000
    return pl.pallas_call(
        _hal_body,
        grid=(N // bn,),
        in_specs=[pl.BlockSpec((bn, D), lambda i: (i, 0)),
                  pl.BlockSpec((H, D, D), lambda i: (0, 0, 0)),
                  pl.BlockSpec((H, D), lambda i: (0, 0)),
                  pl.BlockSpec((H, D), lambda i: (0, 0))],
        out_specs=[pl.BlockSpec((H, bn, D), lambda i: (0, i, 0)),
                   pl.BlockSpec((bn, 2 * H), lambda i: (i, 0))],
        out_shape=[jax.ShapeDtypeStruct((H, N, D), jnp.float32),
                   jax.ShapeDtypeStruct((N, 2 * H), jnp.float32)],
    )(x, gw, gas, gad)


def _sc_edge_agg(h0, h1, als0, ald0, als1, ald1, srcp, dstp):
    mesh = plsc.VectorSubcoreMesh(core_axis_name="c", subcore_axis_name="s")

    @functools.partial(
        pl.kernel,
        mesh=mesh,
        out_type=jax.ShapeDtypeStruct((H, 2 * ROWS, CH), jnp.float32),
        compiler_params=pltpu.CompilerParams(needs_layout_passes=False,
                                             use_tc_tiling_on_sc=False),
        scratch_types=[
            pltpu.VMEM_SHARED((ROWS, CH), jnp.float32),
            pltpu.VMEM((CHUNK,), jnp.int32),
            pltpu.VMEM((CHUNK,), jnp.int32),
            pltpu.VMEM((CHUNK,), jnp.float32),
            pltpu.VMEM((CHUNK,), jnp.float32),
            pltpu.VMEM((CHUNK,), jnp.float32),
            pltpu.VMEM((CHUNK, D), jnp.float32),
            pltpu.VMEM((CHUNK, CH), jnp.float32),
            pltpu.SemaphoreType.DMA,
            pltpu.SemaphoreType.DMA,
        ],
    )
    def k(h0_hbm, h1_hbm, als0_hbm, ald0_hbm, als1_hbm, ald1_hbm,
          src_hbm, dst_hbm, out_hbm,
          acc, src_v, dst_v, alv_v, adv_v, w_v, rows_v, stage_v, sem, sem2):
        cid = lax.axis_index("c")
        sid = lax.axis_index("s")
        wid = cid * 16 + sid
        iota = lax.iota(jnp.int32, LANES)
        rps = ROWS // 16

        for hp in range(H):
            h_hbm = (h0_hbm, h1_hbm)[hp]
            as_hbm = (als0_hbm, als1_hbm)[hp]
            ad_hbm = (ald0_hbm, ald1_hbm)[hp]

            def zb(i, _):
                for dpart in range(CH // LANES):
                    stage_v[i, pl.ds(dpart * LANES, LANES)] = (
                        jnp.zeros((LANES,), jnp.float32))
                return 0
            lax.fori_loop(0, CHUNK, zb, 0)

            def zacc(j, _):
                t = sid + j * 16

                @pl.when(t < ROWS // CHUNK)
                def _():
                    pltpu.sync_copy(stage_v, acc.at[pl.ds(t * CHUNK, CHUNK)])
                return 0
            lax.fori_loop(0, (ROWS // CHUNK + 15) // 16, zacc, 0)
            plsc.subcore_barrier()

            def chunk_body(c, _):
                base = wid * EPT + c * CHUNK
                pltpu.sync_copy(src_hbm.at[pl.ds(base, CHUNK)], src_v)
                pltpu.sync_copy(dst_hbm.at[pl.ds(base, CHUNK)], dst_v)
                gat = pltpu.async_copy(h_hbm.at[src_v], rows_v, sem)
                ga = pltpu.async_copy(as_hbm.at[src_v], alv_v, sem2)
                gb = pltpu.async_copy(ad_hbm.at[dst_v], adv_v, sem2)
                ga.wait()
                gb.wait()
                for g in range(CHUNK // LANES):
                    av = alv_v[pl.ds(g * LANES, LANES)]
                    bv = adv_v[pl.ds(g * LANES, LANES)]
                    xv = av + bv
                    w = jnp.exp(jnp.maximum(xv, 0.2 * xv))
                    w_v[pl.ds(g * LANES, LANES)] = w
                    plsc.store_scatter(
                        stage_v,
                        [g * LANES + iota, jnp.full((LANES,), D, jnp.int32)],
                        w)
                gat.wait()

                def scale(g, _):
                    w16 = w_v[pl.ds(g * LANES, LANES)]
                    for j in range(LANES):
                        wi = w16[j]
                        i = g * LANES + j
                        for dpart in range(D // LANES):
                            v = rows_v[i, pl.ds(dpart * LANES, LANES)]
                            stage_v[i, pl.ds(dpart * LANES, LANES)] = v * wi
                    return 0
                lax.fori_loop(0, CHUNK // LANES, scale, 0)
                pltpu.sync_copy(stage_v, acc.at[dst_v], add=True)
                return 0
            lax.fori_loop(0, NCHUNKS, chunk_body, 0)
            plsc.subcore_barrier()

            pltpu.sync_copy(
                acc.at[pl.ds(sid * rps, rps)],
                out_hbm.at[hp, pl.ds(cid * ROWS + sid * rps, rps)])
            plsc.subcore_barrier()

    return k(h0, h1, als0, ald0, als1, ald1, srcp, dstp)


def _comb_body(a_ref, b_ref, o_ref):
    accm = None
    for h in range(H):
        num = a_ref[h, :, :D] + b_ref[h, :, :D]
        den = a_ref[h, :, D:D + 1] + b_ref[h, :, D:D + 1]
        v = num / (den + 1e-16)
        e = jnp.where(v > 0, v, jnp.exp(jnp.minimum(v, 0.0)) - 1.0)
        accm = e if accm is None else accm + e
    m = accm * (1.0 / H)
    nrm = jnp.sqrt(jnp.sum(m * m, axis=1, keepdims=True))
    o_ref[...] = m / (nrm + 1e-12)


def _combine(agg):
    bn = 720
    return pl.pallas_call(
        _comb_body,
        grid=(ROWS // bn,),
        in_specs=[pl.BlockSpec((H, bn, CH), lambda i: (0, i, 0)),
                  pl.BlockSpec((H, bn, CH), lambda i: (0, ROWS // bn + i, 0))],
        out_specs=pl.BlockSpec((bn, D), lambda i: (i, 0)),
        out_shape=jax.ShapeDtypeStruct((ROWS, D), jnp.float32),
    )(agg, agg)


_INV_SQRT_D = 0.08838834764831845


def _mha_body(x0_ref, x1_ref, x2_ref, wq_ref, wk_ref, wv_ref, o_ref):
    xs = [x0_ref[...], x1_ref[...], x2_ref[...]]
    for h in range(H):
        q = [jnp.dot(x, wq_ref[h], preferred_element_type=jnp.float32)
             for x in xs]
        kk = [jnp.dot(x, wk_ref[h], preferred_element_type=jnp.float32)
              for x in xs]
        vv = [jnp.dot(x, wv_ref[h], preferred_element_type=jnp.float32)
              for x in xs]
        osum = None
        for r in range(R):
            att = [jnp.sum(q[r] * kk[s], axis=1, keepdims=True) * _INV_SQRT_D
                   for s in range(R)]
            m = jnp.maximum(jnp.maximum(att[0], att[1]), att[2])
            ee = [jnp.exp(a - m) for a in att]
            den = ee[0] + ee[1] + ee[2]
            o_r = (ee[0] * vv[0] + ee[1] * vv[1] + ee[2] * vv[2]) / den
            osum = o_r if osum is None else osum + o_r
        o_ref[:, h * D:(h + 1) * D] = osum * (1.0 / R)


def _mha(x0, x1, x2, wq, wk, wv):
    bn = 1000
    return pl.pallas_call(
        _mha_body,
        grid=(N // bn,),
        in_specs=[pl.BlockSpec((bn, D), lambda i: (i, 0)),
                  pl.BlockSpec((bn, D), lambda i: (i, 0)),
                  pl.BlockSpec((bn, D), lambda i: (i, 0)),
                  pl.BlockSpec((H, D, D), lambda i: (0, 0, 0)),
                  pl.BlockSpec((H, D, D), lambda i: (0, 0, 0)),
                  pl.BlockSpec((H, D, D), lambda i: (0, 0, 0))],
        out_specs=pl.BlockSpec((bn, H * D), lambda i: (i, 0)),
        out_shape=jax.ShapeDtypeStruct((N, H * D), jnp.float32),
    )(x0, x1, x2, wq, wk, wv)


def _proj_body(adj_ref, emb_ref, fused_ref, w_ref, b_ref, o_ref):
    adj = adj_ref[...]
    rs = jnp.sum(adj, axis=1, keepdims=True)
    ragg = jnp.dot(adj, emb_ref[...],
                   preferred_element_type=jnp.float32) / (rs + 1e-5)
    f = jnp.dot(fused_ref[...], w_ref[:H * D, :],
                preferred_element_type=jnp.float32)
    g = jnp.dot(ragg, w_ref[H * D:, :], preferred_element_type=jnp.float32)
    o_ref[...] = jnp.maximum(f + g + b_ref[...], 0.0)


def _relproj(rel_adj, rel_emb, fused, proj_w, proj_b2):
    bn = 1000
    return pl.pallas_call(
        _proj_body,
        grid=(N // bn,),
        in_specs=[pl.BlockSpec((bn, RN), lambda i: (i, 0)),
                  pl.BlockSpec((RN, D), lambda i: (0, 0)),
                  pl.BlockSpec((bn, H * D), lambda i: (i, 0)),
                  pl.BlockSpec((H * D + D, D), lambda i: (0, 0)),
                  pl.BlockSpec((1, D), lambda i: (0, 0))],
        out_specs=pl.BlockSpec((bn, D), lambda i: (i, 0)),
        out_shape=jax.ShapeDtypeStruct((N, D), jnp.float32),
    )(rel_adj, rel_emb, fused, proj_w, proj_b2)


def _forward(ent, rel_emb, rel_adj, edge, gat_w, gat_asrc, gat_adst,
             wq, wk, wv, proj_w, proj_b2):
    npad = EPAD - E
    srcp = jnp.concatenate(
        [edge[0].astype(jnp.int32),
         jnp.arange(npad, dtype=jnp.int32) % N])
    dstp = jnp.concatenate(
        [edge[1].astype(jnp.int32),
         N + jnp.arange(npad, dtype=jnp.int32) % (ROWS - N)])
    xs = [ent]
    x = ent
    for l in range(NLAYERS):
        hml, al = _hidden_al(x, gat_w[l], gat_asrc[l], gat_adst[l])
        alp = jnp.pad(al, ((0, NTAB - N), (0, 0)))
        agg = _sc_edge_agg(hml[0], hml[1], alp[:, 0], alp[:, 2],
                           alp[:, 1], alp[:, 3], srcp, dstp)
        x = _combine(agg)[:N]
        xs.append(x)
    fused = _mha(xs[0], xs[1], xs[2], wq, wk, wv)
    return _relproj(rel_adj, rel_emb, fused, proj_w, proj_b2)


def kernel(ent_sr, ent_tg, rel_emb_sr, rel_emb_tg, rel_adj_sr, rel_adj_tg,
           gat_W, gat_asrc, gat_adst, Wq, Wk, Wv, proj_W, proj_b,
           edge_sr, edge_tg):
    pb = proj_b.reshape(1, D)
    sr = _forward(ent_sr, rel_emb_sr, rel_adj_sr, edge_sr,
                  gat_W, gat_asrc, gat_adst, Wq, Wk, Wv, proj_W, pb)
    tg = _forward(ent_tg, rel_emb_tg, rel_adj_tg, edge_tg,
                  gat_W, gat_asrc, gat_adst, Wq, Wk, Wv, proj_W, pb)
    return (sr, tg)
